# baseline (speedup 1.0000x reference)
import functools

import jax
import jax.numpy as jnp
from jax import lax
from jax.experimental import pallas as pl
from jax.experimental.pallas import tpu as pltpu
from jax.experimental.pallas import tpu_sc as plsc


@functools.lru_cache(maxsize=None)
def _make_sc_kernel(n_i: int, n_j: int, n_rows: int, d: int):
    info = plsc.get_sparse_core_info()
    nc, ns = info.num_cores, info.num_subcores
    nw = nc * ns
    jw_n, iw_n = 8, 4
    assert nw == jw_n * iw_n
    assert n_j % jw_n == 0 and n_i % iw_n == 0
    j_per_w = n_j // jw_n
    i_per_w = n_i // iw_n
    chunk = 256
    nbuf = 5
    assert i_per_w % chunk == 0
    ic_per_j = i_per_w // chunk
    n_chunks = j_per_w * ic_per_j
    assert n_chunks % nbuf == 0
    lanes = 16
    vpc = chunk // lanes

    mesh = plsc.VectorSubcoreMesh(core_axis_name="c", subcore_axis_name="s")

    idx_types = [pltpu.VMEM((chunk,), jnp.int32) for _ in range(nbuf)]
    out_types = [pltpu.VMEM((d, chunk), jnp.float32) for _ in range(nbuf)]

    @functools.partial(
        pl.kernel,
        mesh=mesh,
        out_type=jax.ShapeDtypeStruct((n_j, d, n_i), jnp.float32),
        scratch_types=idx_types
        + out_types
        + [
            pltpu.VMEM((n_rows * d,), jnp.float32),
            pltpu.SemaphoreType.DMA((nbuf,)),
            pltpu.SemaphoreType.DMA((nbuf,)),
        ],
        compiler_params=pltpu.CompilerParams(needs_layout_passes=False),
    )
    def k(tab_hbm, idx_hbm, out_hbm, *refs):
        idx_v = refs[:nbuf]
        out_v = refs[nbuf : 2 * nbuf]
        tab_v, sem_i, sem_o = refs[2 * nbuf :]

        cid = lax.axis_index("c")
        sid = lax.axis_index("s")
        wid = sid * nc + cid
        jw = wid // iw_n
        iw = wid - jw * iw_n
        j0 = jw * j_per_w
        i0w = iw * i_per_w

        pltpu.sync_copy(tab_hbm, tab_v)

        def chunk_off(g):
            j = j0 + g // ic_per_j
            i0 = i0w + (g % ic_per_j) * chunk
            return j, i0

        def start_idx(g, b):
            j, i0 = chunk_off(g)
            pltpu.async_copy(
                idx_hbm.at[pl.ds(j * n_i + i0, chunk)], idx_v[b], sem_i.at[b]
            )

        def wait_idx(b):
            pltpu.make_async_copy(
                idx_hbm.at[pl.ds(0, chunk)], idx_v[b], sem_i.at[b]
            ).wait()

        def start_store(g, b):
            j, i0 = chunk_off(g)
            pltpu.async_copy(
                out_v[b], out_hbm.at[j].at[:, pl.ds(i0, chunk)], sem_o.at[b]
            )

        def wait_store(b):
            pltpu.make_async_copy(
                out_v[b], out_hbm.at[0].at[:, pl.ds(0, chunk)], sem_o.at[b]
            ).wait()

        def compute(b):
            def vbody(v, carry):
                iv = idx_v[b][pl.ds(v * lanes, lanes)]
                base = iv * d
                for c in range(d):
                    out_v[b][c, pl.ds(v * lanes, lanes)] = plsc.load_gather(
                        tab_v, [base + c]
                    )
                return carry

            lax.fori_loop(0, vpc, vbody, 0)

        for b in range(nbuf):
            start_idx(b, b)

        def body(go, carry):
            for b in range(nbuf):
                g = go * nbuf + b
                wait_idx(b)

                @pl.when(g >= nbuf)
                def _(b=b):
                    wait_store(b)

                compute(b)
                start_store(g, b)

                @pl.when(g + nbuf < n_chunks)
                def _(g=g, b=b):
                    start_idx(g + nbuf, b)

            return carry

        lax.fori_loop(0, n_chunks // nbuf, body, 0)

        for b in range(nbuf):
            wait_store(b)

    return k


def kernel(time, table):
    n, t = time.shape
    n_rows, d = table.shape
    idx = time.T.reshape(n * t).astype(jnp.int32)
    tab_flat = table.reshape(n_rows * d)
    out3 = _make_sc_kernel(n, t, n_rows, d)(tab_flat, idx)
    return out3.transpose(2, 0, 1)


# speedup vs baseline: 1.6686x; 1.6686x over previous
import functools

import jax
import jax.numpy as jnp
from jax import lax
from jax.experimental import pallas as pl
from jax.experimental.pallas import tpu as pltpu
from jax.experimental.pallas import tpu_sc as plsc


@functools.lru_cache(maxsize=None)
def _make_sc_kernel(n_i: int, n_j: int, n_rows: int, d: int):
    info = plsc.get_sparse_core_info()
    nc, ns = info.num_cores, info.num_subcores
    nw = nc * ns
    jw_n, iw_n = 8, 4
    assert nw == jw_n * iw_n
    assert n_j % jw_n == 0 and n_i % iw_n == 0
    j_per_w = n_j // jw_n
    i_per_w = n_i // iw_n
    chunk = 256
    nbuf = 5
    assert i_per_w % chunk == 0
    ic_per_j = i_per_w // chunk
    n_chunks = j_per_w * ic_per_j
    assert n_chunks % nbuf == 0
    lanes = 16
    vpc = chunk // lanes

    mesh = plsc.VectorSubcoreMesh(core_axis_name="c", subcore_axis_name="s")

    idx_types = [pltpu.VMEM((chunk,), jnp.int32) for _ in range(nbuf)]
    out_types = [pltpu.VMEM((d, chunk), jnp.float32) for _ in range(nbuf)]

    @functools.partial(
        pl.kernel,
        mesh=mesh,
        out_type=jax.ShapeDtypeStruct((n_j, d, n_i), jnp.float32),
        scratch_types=idx_types
        + out_types
        + [
            pltpu.VMEM((n_rows * d,), jnp.float32),
            pltpu.SemaphoreType.DMA((nbuf,)),
            pltpu.SemaphoreType.DMA((nbuf,)),
        ],
        compiler_params=pltpu.CompilerParams(needs_layout_passes=False),
    )
    def k(tab_hbm, idx_hbm, out_hbm, *refs):
        idx_v = refs[:nbuf]
        out_v = refs[nbuf : 2 * nbuf]
        tab_v, sem_i, sem_o = refs[2 * nbuf :]

        cid = lax.axis_index("c")
        sid = lax.axis_index("s")
        wid = sid * nc + cid
        jw = wid // iw_n
        iw = wid - jw * iw_n
        j0 = jw * j_per_w
        i0w = iw * i_per_w

        pltpu.sync_copy(tab_hbm, tab_v)

        def chunk_off(g):
            j = j0 + g // ic_per_j
            i0 = i0w + (g % ic_per_j) * chunk
            return j, i0

        def start_idx(g, b):
            j, i0 = chunk_off(g)
            pltpu.async_copy(
                idx_hbm.at[pl.ds(j * n_i + i0, chunk)], idx_v[b], sem_i.at[b]
            )

        def wait_idx(b):
            pltpu.make_async_copy(
                idx_hbm.at[pl.ds(0, chunk)], idx_v[b], sem_i.at[b]
            ).wait()

        def start_store(g, b):
            j, i0 = chunk_off(g)
            pltpu.async_copy(
                out_v[b], out_hbm.at[j].at[:, pl.ds(i0, chunk)], sem_o.at[b]
            )

        def wait_store(b):
            pltpu.make_async_copy(
                out_v[b], out_hbm.at[0].at[:, pl.ds(0, chunk)], sem_o.at[b]
            ).wait()

        def compute(b):
            @plsc.parallel_loop(0, chunk, lanes, unroll=2)
            def vbody(v0):
                iv = idx_v[b][pl.ds(v0, lanes)]
                base = iv * d
                for c in range(d):
                    out_v[b][c, pl.ds(v0, lanes)] = plsc.load_gather(
                        tab_v, [base + c]
                    )

        for b in range(nbuf):
            start_idx(b, b)

        def body(go, carry):
            for b in range(nbuf):
                g = go * nbuf + b
                wait_idx(b)

                @pl.when(g >= nbuf)
                def _(b=b):
                    wait_store(b)

                compute(b)
                start_store(g, b)

                @pl.when(g + nbuf < n_chunks)
                def _(g=g, b=b):
                    start_idx(g + nbuf, b)

            return carry

        lax.fori_loop(0, n_chunks // nbuf, body, 0)

        for b in range(nbuf):
            wait_store(b)

    return k


def kernel(time, table):
    n, t = time.shape
    n_rows, d = table.shape
    idx = time.T.reshape(n * t).astype(jnp.int32)
    tab_flat = table.reshape(n_rows * d)
    out3 = _make_sc_kernel(n, t, n_rows, d)(tab_flat, idx)
    return out3.transpose(2, 0, 1)


# speedup vs baseline: 1.8122x; 1.0861x over previous
import functools

import jax
import jax.numpy as jnp
from jax import lax
from jax.experimental import pallas as pl
from jax.experimental.pallas import tpu as pltpu
from jax.experimental.pallas import tpu_sc as plsc


@functools.lru_cache(maxsize=None)
def _make_sc_kernel(n_i: int, n_j: int, n_rows: int, d: int):
    info = plsc.get_sparse_core_info()
    nc, ns = info.num_cores, info.num_subcores
    nw = nc * ns
    jw_n, iw_n = 8, 4
    assert nw == jw_n * iw_n
    assert n_j % jw_n == 0 and n_i % iw_n == 0
    j_per_w = n_j // jw_n
    i_per_w = n_i // iw_n
    chunk = 256
    nbuf = 5
    assert i_per_w % chunk == 0
    ic_per_j = i_per_w // chunk
    n_chunks = j_per_w * ic_per_j
    assert n_chunks % nbuf == 0
    lanes = 16
    vpc = chunk // lanes

    mesh = plsc.VectorSubcoreMesh(core_axis_name="c", subcore_axis_name="s")

    idx_types = [pltpu.VMEM((chunk,), jnp.int32) for _ in range(nbuf)]
    out_types = [pltpu.VMEM((d, chunk), jnp.float32) for _ in range(nbuf)]

    @functools.partial(
        pl.kernel,
        mesh=mesh,
        out_type=jax.ShapeDtypeStruct((n_j, d, n_i), jnp.float32),
        scratch_types=idx_types
        + out_types
        + [
            pltpu.VMEM((n_rows * d,), jnp.float32),
            pltpu.SemaphoreType.DMA((nbuf,)),
            pltpu.SemaphoreType.DMA((nbuf,)),
        ],
        compiler_params=pltpu.CompilerParams(needs_layout_passes=False),
    )
    def k(tab_hbm, idx_hbm, out_hbm, *refs):
        idx_v = refs[:nbuf]
        out_v = refs[nbuf : 2 * nbuf]
        tab_v, sem_i, sem_o = refs[2 * nbuf :]

        cid = lax.axis_index("c")
        sid = lax.axis_index("s")
        wid = sid * nc + cid
        jw = wid // iw_n
        iw = wid - jw * iw_n
        j0 = jw * j_per_w
        i0w = iw * i_per_w

        pltpu.sync_copy(tab_hbm, tab_v)

        def chunk_off(g):
            j = j0 + g // ic_per_j
            i0 = i0w + (g % ic_per_j) * chunk
            return j, i0

        def start_idx(g, b):
            j, i0 = chunk_off(g)
            pltpu.async_copy(
                idx_hbm.at[pl.ds(j * n_i + i0, chunk)], idx_v[b], sem_i.at[b]
            )

        def wait_idx(b):
            pltpu.make_async_copy(
                idx_hbm.at[pl.ds(0, chunk)], idx_v[b], sem_i.at[b]
            ).wait()

        def start_store(g, b):
            j, i0 = chunk_off(g)
            pltpu.async_copy(
                out_v[b], out_hbm.at[j].at[:, pl.ds(i0, chunk)], sem_o.at[b]
            )

        def wait_store(b):
            pltpu.make_async_copy(
                out_v[b], out_hbm.at[0].at[:, pl.ds(0, chunk)], sem_o.at[b]
            ).wait()

        def compute(b):
            @plsc.parallel_loop(0, chunk, lanes, unroll=2)
            def vbody(v0):
                iv = idx_v[b][pl.ds(v0, lanes)]
                base = iv * d
                for c0 in range(0, d, 16):
                    vals = [
                        plsc.load_gather(tab_v, [base + c])
                        for c in range(c0, c0 + 16)
                    ]
                    for k, c in enumerate(range(c0, c0 + 16)):
                        out_v[b][c, pl.ds(v0, lanes)] = vals[k]

        for b in range(nbuf):
            start_idx(b, b)

        def body(go, carry):
            for b in range(nbuf):
                g = go * nbuf + b
                wait_idx(b)

                @pl.when(g >= nbuf)
                def _(b=b):
                    wait_store(b)

                compute(b)
                start_store(g, b)

                @pl.when(g + nbuf < n_chunks)
                def _(g=g, b=b):
                    start_idx(g + nbuf, b)

            return carry

        lax.fori_loop(0, n_chunks // nbuf, body, 0)

        for b in range(nbuf):
            wait_store(b)

    return k


def kernel(time, table):
    n, t = time.shape
    n_rows, d = table.shape
    idx = time.T.reshape(n * t).astype(jnp.int32)
    tab_flat = table.reshape(n_rows * d)
    out3 = _make_sc_kernel(n, t, n_rows, d)(tab_flat, idx)
    return out3.transpose(2, 0, 1)


# speedup vs baseline: 1.8158x; 1.0020x over previous
import functools

import jax
import jax.numpy as jnp
from jax import lax
from jax.experimental import pallas as pl
from jax.experimental.pallas import tpu as pltpu
from jax.experimental.pallas import tpu_sc as plsc


@functools.lru_cache(maxsize=None)
def _make_sc_kernel(n_i: int, n_j: int, n_rows: int, d: int):
    info = plsc.get_sparse_core_info()
    nc, ns = info.num_cores, info.num_subcores
    nw = nc * ns
    jw_n, iw_n = 8, 4
    assert nw == jw_n * iw_n
    assert n_j % jw_n == 0 and n_i % iw_n == 0
    j_per_w = n_j // jw_n
    i_per_w = n_i // iw_n
    chunk = 256
    nbuf = 5
    assert i_per_w % chunk == 0
    ic_per_j = i_per_w // chunk
    n_chunks = j_per_w * ic_per_j
    assert n_chunks % nbuf == 0
    lanes = 16
    vpc = chunk // lanes

    mesh = plsc.VectorSubcoreMesh(core_axis_name="c", subcore_axis_name="s")

    idx_types = [pltpu.VMEM((chunk,), jnp.int32) for _ in range(nbuf)]
    out_types = [pltpu.VMEM((d, chunk), jnp.float32) for _ in range(nbuf)]

    @functools.partial(
        pl.kernel,
        mesh=mesh,
        out_type=jax.ShapeDtypeStruct((n_j, d, n_i), jnp.float32),
        scratch_types=idx_types
        + out_types
        + [
            pltpu.VMEM((n_rows * d,), jnp.float32),
            pltpu.SemaphoreType.DMA((nbuf,)),
            pltpu.SemaphoreType.DMA((nbuf,)),
        ],
        compiler_params=pltpu.CompilerParams(
            needs_layout_passes=False, disable_bounds_checks=True
        ),
    )
    def k(tab_hbm, idx_hbm, out_hbm, *refs):
        idx_v = refs[:nbuf]
        out_v = refs[nbuf : 2 * nbuf]
        tab_v, sem_i, sem_o = refs[2 * nbuf :]

        cid = lax.axis_index("c")
        sid = lax.axis_index("s")
        wid = sid * nc + cid
        jw = wid // iw_n
        iw = wid - jw * iw_n
        j0 = jw * j_per_w
        i0w = iw * i_per_w

        pltpu.sync_copy(tab_hbm, tab_v)

        def chunk_off(g):
            j = j0 + g // ic_per_j
            i0 = i0w + (g % ic_per_j) * chunk
            return j, i0

        def start_idx(g, b):
            j, i0 = chunk_off(g)
            pltpu.async_copy(
                idx_hbm.at[pl.ds(j * n_i + i0, chunk)], idx_v[b], sem_i.at[b]
            )

        def wait_idx(b):
            pltpu.make_async_copy(
                idx_hbm.at[pl.ds(0, chunk)], idx_v[b], sem_i.at[b]
            ).wait()

        def start_store(g, b):
            j, i0 = chunk_off(g)
            pltpu.async_copy(
                out_v[b], out_hbm.at[j].at[:, pl.ds(i0, chunk)], sem_o.at[b]
            )

        def wait_store(b):
            pltpu.make_async_copy(
                out_v[b], out_hbm.at[0].at[:, pl.ds(0, chunk)], sem_o.at[b]
            ).wait()

        def compute(b):
            @plsc.parallel_loop(0, chunk, lanes, unroll=2)
            def vbody(v0):
                iv = idx_v[b][pl.ds(v0, lanes)]
                base = iv * d
                for c0 in range(0, d, 16):
                    vals = [
                        plsc.load_gather(tab_v, [base + c])
                        for c in range(c0, c0 + 16)
                    ]
                    for k, c in enumerate(range(c0, c0 + 16)):
                        out_v[b][c, pl.ds(v0, lanes)] = vals[k]

        for b in range(nbuf):
            start_idx(b, b)

        def body(go, carry):
            for b in range(nbuf):
                g = go * nbuf + b
                wait_idx(b)

                @pl.when(g >= nbuf)
                def _(b=b):
                    wait_store(b)

                compute(b)
                start_store(g, b)

                @pl.when(g + nbuf < n_chunks)
                def _(g=g, b=b):
                    start_idx(g + nbuf, b)

            return carry

        lax.fori_loop(0, n_chunks // nbuf, body, 0)

        for b in range(nbuf):
            wait_store(b)

    return k


def kernel(time, table):
    n, t = time.shape
    n_rows, d = table.shape
    idx = time.T.reshape(n * t).astype(jnp.int32)
    tab_flat = table.reshape(n_rows * d)
    out3 = _make_sc_kernel(n, t, n_rows, d)(tab_flat, idx)
    return out3.transpose(2, 0, 1)


# speedup vs baseline: 13.1684x; 7.2522x over previous
import functools

import jax
import jax.numpy as jnp
from jax import lax
from jax.experimental import pallas as pl
from jax.experimental.pallas import tpu as pltpu
from jax.experimental.pallas import tpu_sc as plsc


@functools.lru_cache(maxsize=None)
def _make_sc_kernel(n_i: int, n_j: int, n_rows: int, d: int, tab_len: int):
    info = plsc.get_sparse_core_info()
    nc, ns = info.num_cores, info.num_subcores
    nw = nc * ns
    jw_n, iw_n = 8, 4
    assert nw == jw_n * iw_n
    assert n_j % jw_n == 0 and n_i % iw_n == 0
    j_per_w = n_j // jw_n
    i_per_w = n_i // iw_n
    chunk = 512
    nbuf = 2
    assert i_per_w % chunk == 0
    ic_per_j = i_per_w // chunk
    n_chunks = j_per_w * ic_per_j
    assert n_chunks % nbuf == 0
    lanes = 16

    mesh = plsc.VectorSubcoreMesh(core_axis_name="c", subcore_axis_name="s")

    idx_types = [pltpu.VMEM((chunk,), jnp.int32) for _ in range(nbuf)]
    out_types = [pltpu.VMEM((d, chunk), jnp.float32) for _ in range(nbuf)]

    @functools.partial(
        pl.kernel,
        mesh=mesh,
        out_type=jax.ShapeDtypeStruct((n_j, d, n_i), jnp.float32),
        scratch_types=idx_types
        + out_types
        + [
            pltpu.VMEM((tab_len,), jnp.float32),
            pltpu.SemaphoreType.DMA((nbuf,)),
            pltpu.SemaphoreType.DMA((nbuf,)),
        ],
        compiler_params=pltpu.CompilerParams(
            needs_layout_passes=False, disable_bounds_checks=True
        ),
    )
    def k(tab_hbm, idx_hbm, out_hbm, *refs):
        idx_v = refs[:nbuf]
        out_v = refs[nbuf : 2 * nbuf]
        tab_v, sem_i, sem_o = refs[2 * nbuf :]

        cid = lax.axis_index("c")
        sid = lax.axis_index("s")
        wid = sid * nc + cid
        jw = wid // iw_n
        iw = wid - jw * iw_n
        j0 = jw * j_per_w
        i0w = iw * i_per_w

        pltpu.sync_copy(tab_hbm, tab_v)

        def chunk_off(g):
            j = j0 + g // ic_per_j
            i0 = i0w + (g % ic_per_j) * chunk
            return j, i0

        def start_idx(g, b):
            j, i0 = chunk_off(g)
            pltpu.async_copy(
                idx_hbm.at[pl.ds(j * n_i + i0, chunk)], idx_v[b], sem_i.at[b]
            )

        def wait_idx(b):
            pltpu.make_async_copy(
                idx_hbm.at[pl.ds(0, chunk)], idx_v[b], sem_i.at[b]
            ).wait()

        def start_store(g, b):
            j, i0 = chunk_off(g)
            pltpu.async_copy(
                out_v[b], out_hbm.at[j].at[:, pl.ds(i0, chunk)], sem_o.at[b]
            )

        def wait_store(b):
            pltpu.make_async_copy(
                out_v[b], out_hbm.at[0].at[:, pl.ds(0, chunk)], sem_o.at[b]
            ).wait()

        def compute(b):
            @plsc.parallel_loop(0, chunk, lanes, unroll=4)
            def vbody(v0):
                iv = idx_v[b][pl.ds(v0, lanes)]
                base = iv * (d + 1)
                for c0 in range(0, d, 32):
                    vals = [
                        plsc.load_gather(tab_v, [base + c])
                        for c in range(c0, c0 + 32)
                    ]
                    for k, c in enumerate(range(c0, c0 + 32)):
                        out_v[b][c, pl.ds(v0, lanes)] = vals[k]

        for b in range(nbuf):
            start_idx(b, b)

        def body(go, carry):
            for b in range(nbuf):
                g = go * nbuf + b
                wait_idx(b)

                @pl.when(g >= nbuf)
                def _(b=b):
                    wait_store(b)

                compute(b)
                start_store(g, b)

                @pl.when(g + nbuf < n_chunks)
                def _(g=g, b=b):
                    start_idx(g + nbuf, b)

            return carry

        lax.fori_loop(0, n_chunks // nbuf, body, 0)

        for b in range(nbuf):
            wait_store(b)

    return k


def kernel(time, table):
    n, t = time.shape
    n_rows, d = table.shape
    idx = time.T.reshape(n * t).astype(jnp.int32)
    tab_flat = jnp.pad(table, ((0, 0), (0, 1))).reshape(n_rows * (d + 1))
    tab_len = -(-(n_rows * (d + 1)) // 16) * 16
    tab_flat = jnp.pad(tab_flat, (0, tab_len - tab_flat.shape[0]))
    out3 = _make_sc_kernel(n, t, n_rows, d, tab_len)(tab_flat, idx)
    return out3.transpose(2, 0, 1)
